# Initial kernel scaffold; baseline (speedup 1.0000x reference)
import jax
import jax.numpy as jnp
from jax.experimental import pallas as pl


def kernel(x, edge_index, W0, al0, ar0, W1, al1, ar1, Wout, bout):
    raise NotImplementedError("write your pallas kernel here")


# speedup vs baseline: 20.6669x; 20.6669x over previous
import jax
import jax.numpy as jnp
from jax import lax
from jax.experimental import pallas as pl
from jax.experimental.pallas import tpu as pltpu
from jax.experimental.pallas import tpu_sc as plsc

N = 10000
E = 320000
F = 128
FH = 64
H = 4
NCLASS = 64
NP = 10240
B = 128
NBLK = 157
EPT = NBLK * B
NTILE = 16
EPAD = NTILE * EPT
STRIPE = NP // NTILE
RB = 640
NEG = -1e30


def _proj_body(x_ref, w0_ref, w1_ref, al0_ref, ar0_ref, al1_ref, ar1_ref,
               ff_ref, el_ref, er_ref):
  xb = x_ref[...]
  for l in range(2):
    w_ref = (w0_ref, w1_ref)[l]
    al_ref = (al0_ref, al1_ref)[l]
    ar_ref = (ar0_ref, ar1_ref)[l]
    for h in range(H):
      f = jnp.dot(xb, w_ref[:, h * F:(h + 1) * F],
                  preferred_element_type=jnp.float32)
      ff_ref[2 * (l * H + h)] = f[:, :FH]
      ff_ref[2 * (l * H + h) + 1] = f[:, FH:]
      el_ref[l * H + h, :] = jnp.sum(f * al_ref[h, :][None, :], axis=-1)
      er_ref[l * H + h, :] = jnp.sum(f * ar_ref[h, :][None, :], axis=-1)


def _project(xp, W0, W1, al0, ar0, al1, ar1):
  full2 = lambda i: (0, 0)
  return pl.pallas_call(
      _proj_body,
      grid=(NP // RB,),
      in_specs=[
          pl.BlockSpec((RB, F), lambda i: (i, 0)),
          pl.BlockSpec((F, H * F), full2),
          pl.BlockSpec((F, H * F), full2),
          pl.BlockSpec((H, F), full2),
          pl.BlockSpec((H, F), full2),
          pl.BlockSpec((H, F), full2),
          pl.BlockSpec((H, F), full2),
      ],
      out_specs=[
          pl.BlockSpec((4 * H, RB, FH), lambda i: (0, i, 0)),
          pl.BlockSpec((2 * H, RB), lambda i: (0, i)),
          pl.BlockSpec((2 * H, RB), lambda i: (0, i)),
      ],
      out_shape=[
          jax.ShapeDtypeStruct((4 * H, NP, FH), jnp.float32),
          jax.ShapeDtypeStruct((2 * H, NP), jnp.float32),
          jax.ShapeDtypeStruct((2 * H, NP), jnp.float32),
      ],
  )(xp, W0, W1, al0, ar0, al1, ar1)


def _sc_body(srcs, dsts, ff, elcat, ercat,
             agg,
             src_all, dst_all, tab_a, tab_b, tab_c,
             rowbuf, sbuf, abuf, bidx,
             den0, den1, acc, sem):
  c = lax.axis_index("c")
  s = lax.axis_index("s")
  pltpu.sync_copy(srcs.at[s], src_all)
  pltpu.sync_copy(dsts.at[s], dst_all)

  zero16 = jnp.zeros((16,), jnp.float32)

  def _zero_rowbuf():
    def _zr(r, carry):
      for k in range(FH // 16):
        rowbuf[r, pl.ds(k * 16, 16)] = zero16
      return carry
    lax.fori_loop(0, B, _zr, 0)

  def _zt(i, carry):
    tab_c[pl.ds(i * 16, 16)] = zero16
    return carry
  lax.fori_loop(0, NP // 16, _zt, 0)

  def layer_body(li, carry):
    pltpu.sync_copy(tab_c.at[pl.ds(0, STRIPE)],
                    den0.at[pl.ds(s * STRIPE, STRIPE)])
    pltpu.sync_copy(tab_c.at[pl.ds(0, STRIPE)],
                    den1.at[pl.ds(s * STRIPE, STRIPE)])
    plsc.subcore_barrier()
    for hh in range(2):
      den = (den0, den1)[hh]
      lane = li * H + 2 * c + hh
      pltpu.sync_copy(elcat.at[pl.ds(lane * NP, NP)], tab_a)
      pltpu.sync_copy(ercat.at[pl.ds(lane * NP, NP)], tab_b)

      def blk1(j, carry1):
        for k in range(8):
          sv = src_all[j, pl.ds(k * 16, 16)]
          dv = dst_all[j, pl.ds(k * 16, 16)]
          e = plsc.load_gather(tab_a, [sv]) + plsc.load_gather(tab_b, [dv])
          e = jnp.where(e > 0, e, 0.2 * e)
          sbuf[pl.ds(k * 16, 16)] = jnp.exp(e)
        pltpu.sync_copy(sbuf, den.at[dst_all.at[j]], add=True)
        return carry1
      lax.fori_loop(0, NBLK, blk1, 0)
    plsc.subcore_barrier()

    for hh in range(2):
      den = (den0, den1)[hh]
      lane = li * H + 2 * c + hh
      pltpu.sync_copy(elcat.at[pl.ds(lane * NP, NP)], tab_a)
      pltpu.sync_copy(ercat.at[pl.ds(lane * NP, NP)], tab_b)
      pltpu.sync_copy(den, tab_c)

      def half_body(fh, carry1):
        base = (2 * lane + fh) * NP
        bvec = jnp.full((16,), base, jnp.int32)
        _zero_rowbuf()
        for q in range(STRIPE // B):
          pltpu.sync_copy(rowbuf, acc.at[pl.ds(s * STRIPE + q * B, B)])
        plsc.subcore_barrier()

        def blk2(j, carry2):
          for k in range(8):
            sv = src_all[j, pl.ds(k * 16, 16)]
            dv = dst_all[j, pl.ds(k * 16, 16)]
            e = plsc.load_gather(tab_a, [sv]) + plsc.load_gather(tab_b, [dv])
            e = jnp.where(e > 0, e, 0.2 * e)
            sval = jnp.exp(e)
            dval = plsc.load_gather(tab_c, [dv])
            abuf[pl.ds(k * 16, 16)] = sval / (dval + 1e-16)
            bidx[pl.ds(k * 16, 16)] = sv + bvec
          pltpu.async_copy(ff.at[bidx], rowbuf, sem).wait()
          for r in range(B):
            aspl = plsc.load_gather(abuf, [jnp.full((16,), r, jnp.int32)])
            for k in range(FH // 16):
              rowbuf[r, pl.ds(k * 16, 16)] = (
                  rowbuf[r, pl.ds(k * 16, 16)] * aspl)
          pltpu.sync_copy(rowbuf, acc.at[dst_all.at[j]], add=True)
          return carry2
        lax.fori_loop(0, NBLK, blk2, 0)
        plsc.subcore_barrier()
        row0 = base + s * STRIPE
        pltpu.sync_copy(acc.at[pl.ds(s * STRIPE, STRIPE)],
                        agg.at[pl.ds(row0, STRIPE)])
        plsc.subcore_barrier()
        return carry1
      lax.fori_loop(0, 2, half_body, 0)
    def _zt2(i, carry1):
      tab_c[pl.ds(i * 16, 16)] = zero16
      return carry1
    lax.fori_loop(0, STRIPE // 16, _zt2, 0)
    return carry
  lax.fori_loop(0, 2, layer_body, 0)


def _sc_agg(srcs, dsts, ff, elcat, ercat):
  mesh = plsc.VectorSubcoreMesh(core_axis_name="c", subcore_axis_name="s")
  kfn = pl.kernel(
      _sc_body,
      out_type=jax.ShapeDtypeStruct((4 * H * NP, FH), jnp.float32),
      mesh=mesh,
      compiler_params=pltpu.CompilerParams(needs_layout_passes=False,
                                           use_tc_tiling_on_sc=False),
      scratch_types=[
          pltpu.VMEM((NBLK, B), jnp.int32),
          pltpu.VMEM((NBLK, B), jnp.int32),
          pltpu.VMEM((NP,), jnp.float32),
          pltpu.VMEM((NP,), jnp.float32),
          pltpu.VMEM((NP,), jnp.float32),
          pltpu.VMEM((B, FH), jnp.float32),
          pltpu.VMEM((B,), jnp.float32),
          pltpu.VMEM((B,), jnp.float32),
          pltpu.VMEM((B,), jnp.int32),
          pltpu.VMEM_SHARED((NP,), jnp.float32),
          pltpu.VMEM_SHARED((NP,), jnp.float32),
          pltpu.VMEM_SHARED((NP, FH), jnp.float32),
          pltpu.SemaphoreType.DMA,
      ],
  )
  return kfn(srcs, dsts, ff, elcat, ercat)


def _out_body(agg_ref, wout_ref, bout_ref, o_ref):
  a = agg_ref[...]
  mls = []
  for l in range(2):
    left = (a[8 * l + 0] + a[8 * l + 2] + a[8 * l + 4] + a[8 * l + 6]) * 0.25
    right = (a[8 * l + 1] + a[8 * l + 3] + a[8 * l + 5] + a[8 * l + 7]) * 0.25
    mls.append(jnp.concatenate([left, right], axis=-1))
  h0 = jnp.maximum(mls[0], 0.0)
  h1 = jnp.maximum(h0 + mls[1], 0.0)
  lo = jnp.dot(h1, wout_ref[...], preferred_element_type=jnp.float32)
  lo = lo + bout_ref[...]
  z = lo - jnp.max(lo, axis=-1, keepdims=True)
  o_ref[...] = z - jnp.log(jnp.sum(jnp.exp(z), axis=-1, keepdims=True))


def _readout(agg, Wout, bout2):
  return pl.pallas_call(
      _out_body,
      grid=(NP // RB,),
      in_specs=[
          pl.BlockSpec((4 * H, RB, FH), lambda i: (0, i, 0)),
          pl.BlockSpec((F, NCLASS), lambda i: (0, 0)),
          pl.BlockSpec((1, NCLASS), lambda i: (0, 0)),
      ],
      out_specs=pl.BlockSpec((RB, NCLASS), lambda i: (i, 0)),
      out_shape=jax.ShapeDtypeStruct((NP, NCLASS), jnp.float32),
  )(agg, Wout, bout2)


def kernel(x, edge_index, W0, al0, ar0, W1, al1, ar1, Wout, bout):
  src = edge_index[0].astype(jnp.int32)
  dst = edge_index[1].astype(jnp.int32)
  pad = EPAD - E
  srcs = jnp.concatenate([src, jnp.zeros((pad,), jnp.int32)])
  srcs = srcs.reshape(NTILE, NBLK, B)
  dsts = jnp.concatenate([dst, jnp.full((pad,), N, jnp.int32)])
  dsts = dsts.reshape(NTILE, NBLK, B)
  xp = jnp.pad(x, ((0, NP - N), (0, 0)))

  ff, el, er = _project(xp, W0, W1, al0, ar0, al1, ar1)
  er = er.at[:, N:].set(NEG)

  agg = _sc_agg(srcs, dsts, ff.reshape(4 * H * NP, FH),
                el.reshape(2 * H * NP), er.reshape(2 * H * NP))

  out = _readout(agg.reshape(4 * H, NP, FH), Wout,
                 bout.reshape(1, NCLASS).astype(jnp.float32))
  return out[:N]
